# baseline (speedup 1.0000x reference)
import functools

import jax
import jax.numpy as jnp
from jax import lax
from jax.experimental import pallas as pl
from jax.experimental.pallas import tpu as pltpu
from jax.experimental.pallas import tpu_sc as plsc

B = 4096
S = 200
D = 32
V = 1000000
NC = 2
NS = 16
NW = NC * NS
BW = B // NW
P = 4
NCH = S // P

_mesh = plsc.VectorSubcoreMesh(core_axis_name="c", subcore_axis_name="s")


@functools.partial(
    pl.kernel,
    mesh=_mesh,
    compiler_params=pltpu.CompilerParams(
        use_tc_tiling_on_sc=False, needs_layout_passes=False),
    out_type=jax.ShapeDtypeStruct((NCH, P, 128, 1024), jnp.float32),
    scratch_types=[
        pltpu.VMEM((S, BW), jnp.int32),
        pltpu.VMEM((P * BW, D), jnp.float32),
        pltpu.VMEM((P * BW, D), jnp.float32),
        pltpu.VMEM((4 * P, 1024), jnp.float32),
        pltpu.VMEM((4 * P, 1024), jnp.float32),
        pltpu.VMEM((S, D), jnp.float32),
        pltpu.SemaphoreType.DMA,
        pltpu.SemaphoreType.DMA,
        pltpu.SemaphoreType.DMA,
        pltpu.SemaphoreType.DMA,
    ],
)
def _embed(xT_hbm, tok_hbm, pos_hbm, out_hbm,
           idxT, rows0, rows1, t0, t1, pos_v,
           gsem0, gsem1, ssem0, ssem1):
    wid = lax.axis_index("s") * NC + lax.axis_index("c")

    rows = (rows0, rows1)
    tt = (t0, t1)
    gsem = (gsem0, gsem1)
    ssem = (ssem0, ssem1)

    pltpu.sync_copy(xT_hbm.at[:, pl.ds(wid * BW, BW)], idxT)
    pltpu.sync_copy(pos_hbm, pos_v)

    def start_gather(c, b):
        for p in range(P):
            pltpu.async_copy(
                tok_hbm.at[idxT.at[c * P + p]],
                rows[b].at[pl.ds(p * BW, BW)], gsem[b])

    def wait_gather(b):
        for _ in range(P):
            pltpu.make_async_copy(
                tok_hbm.at[pl.ds(0, BW)],
                rows[b].at[pl.ds(0, BW)], gsem[b]).wait()

    def start_scatter(c, b):
        for g in range(4):
            pltpu.async_copy(
                tt[b].at[pl.ds(g * P, P)],
                out_hbm.at[c, :, g * 32 + wid], ssem[b])

    def wait_scatter(b):
        for _ in range(4):
            pltpu.make_async_copy(
                tt[b].at[pl.ds(0, P)],
                out_hbm.at[0, :, 0], ssem[b]).wait()

    def compute(c, b):
        rv = rows[b]
        tv = tt[b]
        iot = lax.iota(jnp.int32, 16)
        rbase_lo = (iot // 8) * P
        cbase = (iot % 8) * 128

        for p in range(P):
            s = c * P + p
            p_lo = pos_v[s, pl.ds(0, 16)]
            p_hi = pos_v[s, pl.ds(16, 16)]
            ridx_lo = rbase_lo + p
            ridx_hi = ridx_lo + 2 * P

            def body(j8, _, p=p, p_lo=p_lo, p_hi=p_hi,
                     ridx_lo=ridx_lo, ridx_hi=ridx_hi):
                jb = j8 * 8
                for u in range(8):
                    j = jb + u
                    r = p * BW + j
                    cidx = cbase + j
                    lo = rv[r, pl.ds(0, 16)] + p_lo
                    hi = rv[r, pl.ds(16, 16)] + p_hi
                    plsc.store_scatter(tv, [ridx_lo, cidx], lo)
                    plsc.store_scatter(tv, [ridx_hi, cidx], hi)
                return 0

            lax.fori_loop(0, BW // 8, body, 0)

    start_gather(0, 0)
    start_gather(1, 1)

    @pl.loop(0, NCH, step=2)
    def _chunks(c0):
        for b in range(2):
            c = c0 + b

            @pl.when(c0 >= 2)
            def _():
                wait_scatter(b)

            wait_gather(b)
            compute(c, b)
            start_scatter(c, b)

            @pl.when(c0 < NCH - 2)
            def _():
                start_gather(c + 2, b)

    wait_scatter(0)
    wait_scatter(1)


def kernel(x, token_table, pos_table):
    xT = x.T.astype(jnp.int32)
    out = _embed(xT, token_table, pos_table)
    z = out.reshape(S, 4, D, 8, 128)
    z = z.transpose(2, 4, 0, 1, 3)
    return z.reshape(B, S, D)


# speedup vs baseline: 1.0308x; 1.0308x over previous
import functools

import jax
import jax.numpy as jnp
from jax import lax
from jax.experimental import pallas as pl
from jax.experimental.pallas import tpu as pltpu
from jax.experimental.pallas import tpu_sc as plsc

B = 4096
S = 200
D = 32
V = 1000000
NC = 2
NS = 16
NW = NC * NS
TOTAL = B * S
PER_W = TOTAL // NW
R = 800
SEQ_C = R // S
NCH = PER_W // R

_mesh = plsc.VectorSubcoreMesh(core_axis_name="c", subcore_axis_name="s")


@functools.partial(
    pl.kernel,
    mesh=_mesh,
    compiler_params=pltpu.CompilerParams(use_tc_tiling_on_sc=False),
    out_type=jax.ShapeDtypeStruct((TOTAL, D), jnp.float32),
    scratch_types=[
        pltpu.VMEM((PER_W,), jnp.int32),
        pltpu.VMEM((R, D), jnp.float32),
        pltpu.VMEM((R, D), jnp.float32),
        pltpu.VMEM((S, D), jnp.float32),
        pltpu.SemaphoreType.DMA,
        pltpu.SemaphoreType.DMA,
        pltpu.SemaphoreType.DMA,
        pltpu.SemaphoreType.DMA,
    ],
)
def _embed(x_hbm, tok_hbm, pos_hbm, out_hbm,
           idx_v, rows0, rows1, pos_v,
           gsem0, gsem1, ssem0, ssem1):
    wid = lax.axis_index("s") * NC + lax.axis_index("c")
    base = wid * PER_W

    rows = (rows0, rows1)
    gsem = (gsem0, gsem1)
    ssem = (ssem0, ssem1)

    pltpu.sync_copy(x_hbm.at[pl.ds(base, PER_W)], idx_v)
    pltpu.sync_copy(pos_hbm, pos_v)

    def start_gather(c):
        buf = c % 2
        return pltpu.async_copy(
            tok_hbm.at[idx_v.at[pl.ds(c * R, R)]], rows[buf], gsem[buf])

    def add_pos(buf):
        rv = rows[buf]

        def body(p, _):
            lo = pos_v[p, pl.ds(0, 16)]
            hi = pos_v[p, pl.ds(16, 16)]
            for k in range(SEQ_C):
                r = k * S + p
                rv[r, pl.ds(0, 16)] = rv[r, pl.ds(0, 16)] + lo
                rv[r, pl.ds(16, 16)] = rv[r, pl.ds(16, 16)] + hi
            return 0

        lax.fori_loop(0, S, body, 0)

    def start_scatter(c):
        buf = c % 2
        return pltpu.async_copy(
            rows[buf], out_hbm.at[pl.ds(base + c * R, R)], ssem[buf])

    gd = [None, None]
    sd = [None, None]
    gd[0] = start_gather(0)
    for c in range(NCH):
        buf = c % 2
        oth = 1 - buf
        if c + 1 < NCH:
            if sd[oth] is not None:
                sd[oth].wait()
            gd[oth] = start_gather(c + 1)
        gd[buf].wait()
        add_pos(buf)
        sd[buf] = start_scatter(c)
    sd[0].wait()
    sd[1].wait()


def kernel(x, token_table, pos_table):
    xf = x.reshape(-1).astype(jnp.int32)
    out = _embed(xf, token_table, pos_table)
    return out.reshape(B, S, D)
